# baseline (speedup 1.0000x reference)
import functools

import jax
import jax.numpy as jnp
import numpy as np
from jax import lax
from jax.experimental import pallas as pl
from jax.experimental.pallas import tpu as pltpu
from jax.experimental.pallas import tpu_sc as plsc

R_MAX = 5.0
NUM_BESSEL = 8
HIDDEN = 128
N_NODES = 10000
N_EDGES = 160000

_BE_TC = 640
_BN = 400

_NS = 16
_ECHUNK = N_EDGES // 2
_BE_SC = 40
_EPS = _ECHUNK // _NS
_NB_SC = _EPS // _BE_SC
_ACC_ROWS = N_NODES + 48


def _edge_kernel(geoT_ref, w1_ref, w2_ref, w3_ref, w4_ref,
                 we2_ref, m0_ref, m1_ref, m2_ref, m3_ref):
    geoTb = geoT_ref[0]
    rT = geoTb[0:1, :]
    nrow = (lax.broadcasted_iota(jnp.int32, (NUM_BESSEL, _BE_TC), 0)
            .astype(jnp.float32) + 1.0)
    arg = nrow * (np.pi / R_MAX) * rT
    pref = np.sqrt(2.0 / R_MAX)
    besselT = pref * jnp.sin(arg) / rT
    u = rT * (1.0 / R_MAX)
    u2 = u * u
    u4 = u2 * u2
    u6 = u4 * u2
    u7 = u6 * u
    u8 = u7 * u
    env = 1.0 - 28.0 * u6 + 48.0 * u7 - 21.0 * u8
    env = jnp.where(u < 1.0, env, 0.0)
    efT = besselT * env

    def _silu(x):
        return x / (1.0 + jnp.exp(-x))

    h = _silu(lax.dot_general(efT, w1_ref[...],
                              (((0,), (0,)), ((), ())),
                              preferred_element_type=jnp.float32))
    h = _silu(jnp.dot(h, w2_ref[...], preferred_element_type=jnp.float32))
    h = _silu(jnp.dot(h, w3_ref[...], preferred_element_type=jnp.float32))
    tp = jnp.dot(h, w4_ref[...], preferred_element_type=jnp.float32)

    gem = lax.dot_general(geoTb, jnp.eye(8, dtype=jnp.float32),
                          (((0,), (0,)), ((), ())),
                          preferred_element_type=jnp.float32)
    lane = lax.broadcasted_iota(jnp.int32, (_BE_TC, 8), 1).astype(jnp.float32)
    oh = (gem[:, 4:5] == lane).astype(jnp.float32)
    nfup = jnp.dot(oh, we2_ref[...],
                   preferred_element_type=jnp.float32)
    a = nfup * tp[:, :HIDDEN]
    b = nfup * tp[:, HIDDEN:]
    s3 = np.sqrt(3.0)
    m0_ref[...] = a
    m1_ref[...] = (s3 * gem[:, 1:2]) * b
    m2_ref[...] = (s3 * gem[:, 2:3]) * b
    m3_ref[...] = (s3 * gem[:, 3:4]) * b


def _edge_stage(geoT, W1, W2, W3, W4, We2p, off):
    grid = (_ECHUNK // _BE_TC,)
    eb = pl.BlockSpec((_BE_TC, HIDDEN), lambda i: (i, 0))
    outs = pl.pallas_call(
        _edge_kernel,
        grid=grid,
        in_specs=[
            pl.BlockSpec((1, 8, _BE_TC), lambda i: (i + off, 0, 0)),
            pl.BlockSpec((NUM_BESSEL, 64), lambda i: (0, 0)),
            pl.BlockSpec((64, 64), lambda i: (0, 0)),
            pl.BlockSpec((64, 64), lambda i: (0, 0)),
            pl.BlockSpec((64, 2 * HIDDEN), lambda i: (0, 0)),
            pl.BlockSpec((8, HIDDEN), lambda i: (0, 0)),
        ],
        out_specs=[eb, eb, eb, eb],
        out_shape=[jax.ShapeDtypeStruct((_ECHUNK, HIDDEN), jnp.float32)] * 4,
    )(geoT, W1, W2, W3, W4, We2p)
    return outs


_GB = 640
_NGB = N_EDGES // _GB
_NW = 32
_GIT = (_NGB + _NW - 1) // _NW


def _sc_gather_body(pos_ref, an_ref, snd_ref, rcv_ref, geoT_ref,
                    pos_v, an_v, sv, rv, gT, pos_sh, an_sh):
    cid = lax.axis_index("c")
    sid = lax.axis_index("s")
    w = sid * 2 + cid

    @pl.when(sid == 0)
    def _():
        pltpu.sync_copy(pos_ref, pos_sh)
        pltpu.sync_copy(an_ref, an_sh)
    plsc.subcore_barrier()
    pltpu.sync_copy(pos_sh, pos_v)
    pltpu.sync_copy(an_sh, an_v)

    def body(b, carry):
        blk = jnp.minimum(w + _NW * b, _NGB - 1)
        base = pl.multiple_of(blk * _GB, 128)
        pltpu.sync_copy(snd_ref.at[pl.ds(base, _GB)], sv)
        pltpu.sync_copy(rcv_ref.at[pl.ds(base, _GB)], rv)
        del base
        for k in range(_GB // 16):
            s16 = sv[pl.ds(k * 16, 16)]
            r16 = rv[pl.ds(k * 16, 16)]
            s4 = s16 * 4
            d4 = r16 * 4
            xs = plsc.load_gather(pos_v, [s4])
            ys = plsc.load_gather(pos_v, [s4 + 1])
            zs_ = plsc.load_gather(pos_v, [s4 + 2])
            xr = plsc.load_gather(pos_v, [d4])
            yr = plsc.load_gather(pos_v, [d4 + 1])
            zr = plsc.load_gather(pos_v, [d4 + 2])
            dx = xr - xs
            dy = yr - ys
            dz = zr - zs_
            r2 = dx * dx + dy * dy + dz * dz + 1e-9
            iy = jnp.int32(0x5F3759DF) - (
                lax.bitcast_convert_type(r2, jnp.int32) >> 1)
            y = lax.bitcast_convert_type(iy, jnp.float32)
            for _ in range(3):
                y = y * (1.5 - 0.5 * r2 * y * y)
            elem = plsc.load_gather(an_v, [s16]).astype(jnp.float32)
            gT[0, pl.ds(k * 16, 16)] = r2 * y
            gT[1, pl.ds(k * 16, 16)] = dx * y
            gT[2, pl.ds(k * 16, 16)] = dy * y
            gT[3, pl.ds(k * 16, 16)] = dz * y
            gT[4, pl.ds(k * 16, 16)] = elem
        pltpu.sync_copy(gT, geoT_ref.at[blk])
        return carry

    lax.fori_loop(0, _GIT, body, 0)


def _sc_gather(pos4, atomic_numbers, sender, receiver):
    mesh = plsc.VectorSubcoreMesh(core_axis_name="c", subcore_axis_name="s")
    f = pl.kernel(
        _sc_gather_body,
        out_type=jax.ShapeDtypeStruct((_NGB, 8, _GB), jnp.float32),
        mesh=mesh,
        scratch_types=[
            pltpu.VMEM((4 * N_NODES,), jnp.float32),
            pltpu.VMEM((N_NODES,), jnp.int32),
            pltpu.VMEM((_GB,), jnp.int32),
            pltpu.VMEM((_GB,), jnp.int32),
            pltpu.VMEM((8, _GB), jnp.float32),
            pltpu.VMEM_SHARED((4 * N_NODES,), jnp.float32),
            pltpu.VMEM_SHARED((N_NODES,), jnp.int32),
        ],
        compiler_params=pltpu.CompilerParams(needs_layout_passes=False),
    )
    return f(pos4, atomic_numbers, sender, receiver)


def _sc_scatter_body(recv_ref, m0, m1, m2, m3, o0, o1, o2, o3,
                     idx_a, rows_a, idx_b, rows_b, zbuf, acc,
                     sem_a, sem_b, sem_sa, sem_sb):
    cid = lax.axis_index("c")
    sid = lax.axis_index("s")

    zeros16 = jnp.zeros((16,), jnp.float32)
    for i in range(16):
        for j in range(HIDDEN // 16):
            zbuf[i, pl.ds(j * 16, 16)] = zeros16

    def run_chunk(m_hbm, o_hbm):
        def zbody(b, carry):
            row = jnp.minimum(sid * 632 + b * 16, _ACC_ROWS - 16)
            row = pl.multiple_of(row, 8)
            pltpu.sync_copy(zbuf, acc.at[pl.ds(row, 16)])
            return carry

        lax.fori_loop(0, 632 // 16 + 1, zbody, 0)
        plsc.subcore_barrier()

        def start_in(idx_p, rows_p, sem, b):
            base = pl.multiple_of(sid * _EPS + b * _BE_SC, 8)
            pltpu.async_copy(recv_ref.at[pl.ds(base, _BE_SC)], idx_p, sem)
            pltpu.async_copy(m_hbm.at[pl.ds(base, _BE_SC)], rows_p, sem)

        def wait_in(idx_p, rows_p, sem):
            pltpu.make_async_copy(
                recv_ref.at[pl.ds(0, _BE_SC)], idx_p, sem).wait()
            pltpu.make_async_copy(
                m_hbm.at[pl.ds(0, _BE_SC)], rows_p, sem).wait()

        start_in(idx_a, rows_a, sem_a, 0)

        def pbody(j, carry):
            wait_in(idx_a, rows_a, sem_a)
            start_in(idx_b, rows_b, sem_b, 2 * j + 1)
            sca = pltpu.async_copy(rows_a, acc.at[idx_a], sem_sa, add=True)
            wait_in(idx_b, rows_b, sem_b)
            sca.wait()
            start_in(idx_a, rows_a, sem_a, 2 * j + 2)
            scb = pltpu.async_copy(rows_b, acc.at[idx_b], sem_sb, add=True)
            scb.wait()
            return carry

        lax.fori_loop(0, (_NB_SC - 1) // 2, pbody, 0)
        wait_in(idx_a, rows_a, sem_a)
        pltpu.sync_copy(rows_a, acc.at[idx_a], add=True)
        plsc.subcore_barrier()
        row = sid * 624

        @pl.when(sid < _NS - 1)
        def _():
            for c in range(15):
                pltpu.sync_copy(acc.at[pl.ds(row + c * 40, 40)], rows_a)
                pltpu.sync_copy(rows_a, o_hbm.at[pl.ds(row + c * 40, 40)])
            pltpu.sync_copy(acc.at[pl.ds(row + 600, 24)],
                            rows_a.at[pl.ds(0, 24)])
            pltpu.sync_copy(rows_a.at[pl.ds(0, 24)],
                            o_hbm.at[pl.ds(row + 600, 24)])

        @pl.when(sid == _NS - 1)
        def _():
            for c in range(16):
                pltpu.sync_copy(acc.at[pl.ds(row + c * 40, 40)], rows_a)
                pltpu.sync_copy(rows_a, o_hbm.at[pl.ds(row + c * 40, 40)])
        plsc.subcore_barrier()

    @pl.when(cid == 0)
    def _():
        run_chunk(m0, o0)
        run_chunk(m1, o1)

    @pl.when(cid == 1)
    def _():
        run_chunk(m2, o2)
        run_chunk(m3, o3)


def _sc_scatter(recv, m0, m1, m2, m3):
    mesh = plsc.VectorSubcoreMesh(core_axis_name="c", subcore_axis_name="s")
    out_t = [jax.ShapeDtypeStruct((N_NODES, HIDDEN), jnp.float32)] * 4
    f = pl.kernel(
        _sc_scatter_body,
        out_type=out_t,
        mesh=mesh,
        scratch_types=[
            pltpu.VMEM((_BE_SC,), jnp.int32),
            pltpu.VMEM((_BE_SC, HIDDEN), jnp.float32),
            pltpu.VMEM((_BE_SC,), jnp.int32),
            pltpu.VMEM((_BE_SC, HIDDEN), jnp.float32),
            pltpu.VMEM((16, HIDDEN), jnp.float32),
            pltpu.VMEM_SHARED((_ACC_ROWS, HIDDEN), jnp.float32),
            pltpu.SemaphoreType.DMA,
            pltpu.SemaphoreType.DMA,
            pltpu.SemaphoreType.DMA,
            pltpu.SemaphoreType.DMA,
        ],
    )
    return f(recv, m0, m1, m2, m3)


def _node_kernel(m0_ref, m1_ref, m2_ref, m3_ref,
                 n0_ref, n1_ref, n2_ref, n3_ref, ohn_ref,
                 wp0_ref, wp1_ref, wp2_ref, wout_ref, out_ref):
    m0 = m0_ref[...] + n0_ref[...]
    m1 = m1_ref[...] + n1_ref[...]
    m2 = m2_ref[...] + n2_ref[...]
    m3 = m3_ref[...] + n3_ref[...]
    oh = ohn_ref[...]
    w0 = jnp.dot(oh, wp0_ref[...], preferred_element_type=jnp.float32)
    w1 = jnp.dot(oh, wp1_ref[...], preferred_element_type=jnp.float32)
    w2 = jnp.dot(oh, wp2_ref[...], preferred_element_type=jnp.float32)
    s1 = m0
    s2 = m0 * m0 + m1 * m1 + m2 * m2 + m3 * m3
    s3 = s1 * s2
    out_scalar = w0 * s1 + w1 * s2 + w2 * s3
    out_ref[...] = jnp.dot(out_scalar, wout_ref[...],
                           preferred_element_type=jnp.float32)


def _node_stage(ms, ns, onehot_n, Wp_pad, W_out):
    grid = (N_NODES // _BN,)
    nb = pl.BlockSpec((_BN, HIDDEN), lambda i: (i, 0))
    wb = pl.BlockSpec((8, HIDDEN), lambda i: (0, 0))
    return pl.pallas_call(
        _node_kernel,
        grid=grid,
        in_specs=[nb] * 8 + [
                  pl.BlockSpec((_BN, 8), lambda i: (i, 0)),
                  wb, wb, wb,
                  pl.BlockSpec((HIDDEN, HIDDEN), lambda i: (0, 0))],
        out_specs=nb,
        out_shape=jax.ShapeDtypeStruct((N_NODES, HIDDEN), jnp.float32),
    )(*ms, *ns, onehot_n, Wp_pad[0], Wp_pad[1], Wp_pad[2], W_out)


def kernel(positions, atomic_numbers, edge_index, W_emb, W_up,
           W1, W2, W3, W4, W_lin, Wp, W_out):
    sender = edge_index[0]
    receiver = edge_index[1]
    pos4 = jnp.pad(positions, ((0, 0), (0, 1))).reshape(-1)
    geoT = _sc_gather(pos4, atomic_numbers, sender, receiver)

    We2p = jnp.pad(W_emb @ W_up, ((0, 3), (0, 0)))
    W4 = jnp.concatenate([W4[:, 0::2], W4[:, 1::2]], axis=1)

    recv = receiver.astype(jnp.int32)
    mc0 = _edge_stage(geoT, W1, W2, W3, W4, We2p, 0)
    sc0 = _sc_scatter(recv[:_ECHUNK], *mc0)
    mc1 = _edge_stage(geoT, W1, W2, W3, W4, We2p, _ECHUNK // _BE_TC)
    sc1 = _sc_scatter(recv[_ECHUNK:], *mc1)

    onehot_n = jax.nn.one_hot(atomic_numbers, 8, dtype=jnp.float32)
    Wp_pad = jnp.pad(Wp, ((0, 0), (0, 3), (0, 0)))
    return _node_stage(sc0, sc1, onehot_n, Wp_pad, W_out)


# speedup vs baseline: 1.2238x; 1.2238x over previous
import functools

import jax
import jax.numpy as jnp
import numpy as np
from jax import lax
from jax.experimental import pallas as pl
from jax.experimental.pallas import tpu as pltpu
from jax.experimental.pallas import tpu_sc as plsc

R_MAX = 5.0
NUM_BESSEL = 8
HIDDEN = 128
N_NODES = 10000
N_EDGES = 160000

_BE_TC = 640
_BN = 400

_NS = 16
_BE_SC = 80
_C0 = 96000
_C1 = N_EDGES - _C0
_ACC_ROWS = N_NODES + 48


def _edge_kernel(geoT_ref, w1_ref, w2_ref, w3_ref, w4_ref,
                 we2_ref, m0_ref, m1_ref, m2_ref, m3_ref):
    geoTb = geoT_ref[0]
    rT = geoTb[0:1, :]
    nrow = (lax.broadcasted_iota(jnp.int32, (NUM_BESSEL, _BE_TC), 0)
            .astype(jnp.float32) + 1.0)
    arg = nrow * (np.pi / R_MAX) * rT
    pref = np.sqrt(2.0 / R_MAX)
    besselT = pref * jnp.sin(arg) / rT
    u = rT * (1.0 / R_MAX)
    u2 = u * u
    u4 = u2 * u2
    u6 = u4 * u2
    u7 = u6 * u
    u8 = u7 * u
    env = 1.0 - 28.0 * u6 + 48.0 * u7 - 21.0 * u8
    env = jnp.where(u < 1.0, env, 0.0)
    efT = besselT * env

    def _silu(x):
        return x / (1.0 + jnp.exp(-x))

    h = _silu(lax.dot_general(efT, w1_ref[...],
                              (((0,), (0,)), ((), ())),
                              preferred_element_type=jnp.float32))
    h = _silu(jnp.dot(h, w2_ref[...], preferred_element_type=jnp.float32))
    h = _silu(jnp.dot(h, w3_ref[...], preferred_element_type=jnp.float32))
    tp = jnp.dot(h, w4_ref[...], preferred_element_type=jnp.float32)

    gem = lax.dot_general(geoTb, jnp.eye(8, dtype=jnp.float32),
                          (((0,), (0,)), ((), ())),
                          preferred_element_type=jnp.float32)
    lane = lax.broadcasted_iota(jnp.int32, (_BE_TC, 8), 1).astype(jnp.float32)
    oh = (gem[:, 4:5] == lane).astype(jnp.float32)
    nfup = jnp.dot(oh, we2_ref[...],
                   preferred_element_type=jnp.float32)
    a = nfup * tp[:, :HIDDEN]
    b = nfup * tp[:, HIDDEN:]
    s3 = np.sqrt(3.0)
    m0_ref[...] = a
    m1_ref[...] = (s3 * gem[:, 1:2]) * b
    m2_ref[...] = (s3 * gem[:, 2:3]) * b
    m3_ref[...] = (s3 * gem[:, 3:4]) * b


def _edge_stage(geoT, W1, W2, W3, W4, We2p, off, echunk):
    grid = (echunk // _BE_TC,)
    eb = pl.BlockSpec((_BE_TC, HIDDEN), lambda i: (i, 0))
    outs = pl.pallas_call(
        _edge_kernel,
        grid=grid,
        in_specs=[
            pl.BlockSpec((1, 8, _BE_TC), lambda i: (i + off, 0, 0)),
            pl.BlockSpec((NUM_BESSEL, 64), lambda i: (0, 0)),
            pl.BlockSpec((64, 64), lambda i: (0, 0)),
            pl.BlockSpec((64, 64), lambda i: (0, 0)),
            pl.BlockSpec((64, 2 * HIDDEN), lambda i: (0, 0)),
            pl.BlockSpec((8, HIDDEN), lambda i: (0, 0)),
        ],
        out_specs=[eb, eb, eb, eb],
        out_shape=[jax.ShapeDtypeStruct((echunk, HIDDEN), jnp.float32)] * 4,
    )(geoT, W1, W2, W3, W4, We2p)
    return outs


_GB = 640
_NGB = N_EDGES // _GB
_NW = 32
_GIT = (_NGB + _NW - 1) // _NW


def _sc_gather_body(pos_ref, an_ref, snd_ref, rcv_ref, geoT_ref,
                    pos_v, an_v, sv, rv, gT, pos_sh, an_sh):
    cid = lax.axis_index("c")
    sid = lax.axis_index("s")
    w = sid * 2 + cid

    @pl.when(sid == 0)
    def _():
        pltpu.sync_copy(pos_ref, pos_sh)
        pltpu.sync_copy(an_ref, an_sh)
    plsc.subcore_barrier()
    pltpu.sync_copy(pos_sh, pos_v)
    pltpu.sync_copy(an_sh, an_v)

    def body(b, carry):
        blk = jnp.minimum(w + _NW * b, _NGB - 1)
        base = pl.multiple_of(blk * _GB, 128)
        pltpu.sync_copy(snd_ref.at[pl.ds(base, _GB)], sv)
        pltpu.sync_copy(rcv_ref.at[pl.ds(base, _GB)], rv)
        del base
        for k in range(_GB // 16):
            s16 = sv[pl.ds(k * 16, 16)]
            r16 = rv[pl.ds(k * 16, 16)]
            s4 = s16 * 4
            d4 = r16 * 4
            xs = plsc.load_gather(pos_v, [s4])
            ys = plsc.load_gather(pos_v, [s4 + 1])
            zs_ = plsc.load_gather(pos_v, [s4 + 2])
            xr = plsc.load_gather(pos_v, [d4])
            yr = plsc.load_gather(pos_v, [d4 + 1])
            zr = plsc.load_gather(pos_v, [d4 + 2])
            dx = xr - xs
            dy = yr - ys
            dz = zr - zs_
            r2 = dx * dx + dy * dy + dz * dz + 1e-9
            iy = jnp.int32(0x5F3759DF) - (
                lax.bitcast_convert_type(r2, jnp.int32) >> 1)
            y = lax.bitcast_convert_type(iy, jnp.float32)
            for _ in range(3):
                y = y * (1.5 - 0.5 * r2 * y * y)
            elem = plsc.load_gather(an_v, [s16]).astype(jnp.float32)
            gT[0, pl.ds(k * 16, 16)] = r2 * y
            gT[1, pl.ds(k * 16, 16)] = dx * y
            gT[2, pl.ds(k * 16, 16)] = dy * y
            gT[3, pl.ds(k * 16, 16)] = dz * y
            gT[4, pl.ds(k * 16, 16)] = elem
        pltpu.sync_copy(gT, geoT_ref.at[blk])
        return carry

    lax.fori_loop(0, _GIT, body, 0)


def _sc_gather(pos4, atomic_numbers, sender, receiver):
    mesh = plsc.VectorSubcoreMesh(core_axis_name="c", subcore_axis_name="s")
    f = pl.kernel(
        _sc_gather_body,
        out_type=jax.ShapeDtypeStruct((_NGB, 8, _GB), jnp.float32),
        mesh=mesh,
        scratch_types=[
            pltpu.VMEM((4 * N_NODES,), jnp.float32),
            pltpu.VMEM((N_NODES,), jnp.int32),
            pltpu.VMEM((_GB,), jnp.int32),
            pltpu.VMEM((_GB,), jnp.int32),
            pltpu.VMEM((8, _GB), jnp.float32),
            pltpu.VMEM_SHARED((4 * N_NODES,), jnp.float32),
            pltpu.VMEM_SHARED((N_NODES,), jnp.int32),
        ],
        compiler_params=pltpu.CompilerParams(needs_layout_passes=False),
    )
    return f(pos4, atomic_numbers, sender, receiver)


def _sc_scatter_body(eps, nb, recv_ref, zeros_ref, m0, m1, m2, m3,
                     o0, o1, o2, o3,
                     idx_a, rows_a, idx_b, rows_b, acc,
                     sem_a, sem_b, sem_sa, sem_sb):
    cid = lax.axis_index("c")
    sid = lax.axis_index("s")

    def run_chunk(m_hbm, o_hbm):
        row = sid * 632

        @pl.when(sid < _NS - 1)
        def _():
            pltpu.sync_copy(zeros_ref.at[pl.ds(row, 632)],
                            acc.at[pl.ds(row, 632)])

        @pl.when(sid == _NS - 1)
        def _():
            pltpu.sync_copy(zeros_ref.at[pl.ds(row, _ACC_ROWS - 15 * 632)],
                            acc.at[pl.ds(row, _ACC_ROWS - 15 * 632)])
        plsc.subcore_barrier()

        def start_in(idx_p, rows_p, sem, b):
            base = pl.multiple_of(sid * eps + b * _BE_SC, 8)
            pltpu.async_copy(recv_ref.at[pl.ds(base, _BE_SC)], idx_p, sem)
            pltpu.async_copy(m_hbm.at[pl.ds(base, _BE_SC)], rows_p, sem)

        def wait_in(idx_p, rows_p, sem):
            pltpu.make_async_copy(
                recv_ref.at[pl.ds(0, _BE_SC)], idx_p, sem).wait()
            pltpu.make_async_copy(
                m_hbm.at[pl.ds(0, _BE_SC)], rows_p, sem).wait()

        npairs = (nb - 1) // 2
        start_in(idx_a, rows_a, sem_a, 0)

        def pbody(j, carry):
            wait_in(idx_a, rows_a, sem_a)
            start_in(idx_b, rows_b, sem_b, 2 * j + 1)
            sca = pltpu.async_copy(rows_a, acc.at[idx_a], sem_sa, add=True)
            wait_in(idx_b, rows_b, sem_b)
            sca.wait()
            start_in(idx_a, rows_a, sem_a, 2 * j + 2)
            scb = pltpu.async_copy(rows_b, acc.at[idx_b], sem_sb, add=True)
            scb.wait()
            return carry

        lax.fori_loop(0, npairs, pbody, 0)
        wait_in(idx_a, rows_a, sem_a)
        if nb % 2 == 0:
            start_in(idx_b, rows_b, sem_b, nb - 1)
            pltpu.sync_copy(rows_a, acc.at[idx_a], add=True)
            wait_in(idx_b, rows_b, sem_b)
            pltpu.sync_copy(rows_b, acc.at[idx_b], add=True)
        else:
            pltpu.sync_copy(rows_a, acc.at[idx_a], add=True)
        plsc.subcore_barrier()
        wrow = sid * 624

        @pl.when(sid < _NS - 1)
        def _():
            pltpu.sync_copy(acc.at[pl.ds(wrow, 624)],
                            o_hbm.at[pl.ds(wrow, 624)])

        @pl.when(sid == _NS - 1)
        def _():
            pltpu.sync_copy(acc.at[pl.ds(wrow, 640)],
                            o_hbm.at[pl.ds(wrow, 640)])
        plsc.subcore_barrier()

    @pl.when(cid == 0)
    def _():
        run_chunk(m0, o0)
        run_chunk(m1, o1)

    @pl.when(cid == 1)
    def _():
        run_chunk(m2, o2)
        run_chunk(m3, o3)


def _sc_scatter(recv, zeros_hbm, m0, m1, m2, m3, echunk):
    eps = echunk // _NS
    nb = eps // _BE_SC
    mesh = plsc.VectorSubcoreMesh(core_axis_name="c", subcore_axis_name="s")
    out_t = [jax.ShapeDtypeStruct((N_NODES, HIDDEN), jnp.float32)] * 4
    f = pl.kernel(
        functools.partial(_sc_scatter_body, eps, nb),
        out_type=out_t,
        mesh=mesh,
        scratch_types=[
            pltpu.VMEM((_BE_SC,), jnp.int32),
            pltpu.VMEM((_BE_SC, HIDDEN), jnp.float32),
            pltpu.VMEM((_BE_SC,), jnp.int32),
            pltpu.VMEM((_BE_SC, HIDDEN), jnp.float32),
            pltpu.VMEM_SHARED((_ACC_ROWS, HIDDEN), jnp.float32),
            pltpu.SemaphoreType.DMA,
            pltpu.SemaphoreType.DMA,
            pltpu.SemaphoreType.DMA,
            pltpu.SemaphoreType.DMA,
        ],
    )
    return f(recv, zeros_hbm, m0, m1, m2, m3)


def _node_kernel(m0_ref, m1_ref, m2_ref, m3_ref,
                 n0_ref, n1_ref, n2_ref, n3_ref, ohn_ref,
                 wp0_ref, wp1_ref, wp2_ref, wout_ref, out_ref):
    m0 = m0_ref[...] + n0_ref[...]
    m1 = m1_ref[...] + n1_ref[...]
    m2 = m2_ref[...] + n2_ref[...]
    m3 = m3_ref[...] + n3_ref[...]
    oh = ohn_ref[...]
    w0 = jnp.dot(oh, wp0_ref[...], preferred_element_type=jnp.float32)
    w1 = jnp.dot(oh, wp1_ref[...], preferred_element_type=jnp.float32)
    w2 = jnp.dot(oh, wp2_ref[...], preferred_element_type=jnp.float32)
    s1 = m0
    s2 = m0 * m0 + m1 * m1 + m2 * m2 + m3 * m3
    s3 = s1 * s2
    out_scalar = w0 * s1 + w1 * s2 + w2 * s3
    out_ref[...] = jnp.dot(out_scalar, wout_ref[...],
                           preferred_element_type=jnp.float32)


def _node_stage(ms, ns, onehot_n, Wp_pad, W_out):
    grid = (N_NODES // _BN,)
    nb = pl.BlockSpec((_BN, HIDDEN), lambda i: (i, 0))
    wb = pl.BlockSpec((8, HIDDEN), lambda i: (0, 0))
    return pl.pallas_call(
        _node_kernel,
        grid=grid,
        in_specs=[nb] * 8 + [
                  pl.BlockSpec((_BN, 8), lambda i: (i, 0)),
                  wb, wb, wb,
                  pl.BlockSpec((HIDDEN, HIDDEN), lambda i: (0, 0))],
        out_specs=nb,
        out_shape=jax.ShapeDtypeStruct((N_NODES, HIDDEN), jnp.float32),
    )(*ms, *ns, onehot_n, Wp_pad[0], Wp_pad[1], Wp_pad[2], W_out)


def kernel(positions, atomic_numbers, edge_index, W_emb, W_up,
           W1, W2, W3, W4, W_lin, Wp, W_out):
    sender = edge_index[0]
    receiver = edge_index[1]
    pos4 = jnp.pad(positions, ((0, 0), (0, 1))).reshape(-1)
    geoT = _sc_gather(pos4, atomic_numbers, sender, receiver)

    We2p = jnp.pad(W_emb @ W_up, ((0, 3), (0, 0)))
    W4 = jnp.concatenate([W4[:, 0::2], W4[:, 1::2]], axis=1)

    recv = receiver.astype(jnp.int32)
    zeros_hbm = jnp.zeros((_ACC_ROWS, HIDDEN), jnp.float32)
    mc0 = _edge_stage(geoT, W1, W2, W3, W4, We2p, 0, _C0)
    sc0 = _sc_scatter(recv[:_C0], zeros_hbm, *mc0, echunk=_C0)
    mc1 = _edge_stage(geoT, W1, W2, W3, W4, We2p, _C0 // _BE_TC, _C1)
    sc1 = _sc_scatter(recv[_C0:], zeros_hbm, *mc1, echunk=_C1)

    onehot_n = jax.nn.one_hot(atomic_numbers, 8, dtype=jnp.float32)
    Wp_pad = jnp.pad(Wp, ((0, 0), (0, 3), (0, 0)))
    return _node_stage(sc0, sc1, onehot_n, Wp_pad, W_out)


# speedup vs baseline: 1.3108x; 1.0711x over previous
import functools

import jax
import jax.numpy as jnp
import numpy as np
from jax import lax
from jax.experimental import pallas as pl
from jax.experimental.pallas import tpu as pltpu
from jax.experimental.pallas import tpu_sc as plsc

R_MAX = 5.0
NUM_BESSEL = 8
HIDDEN = 128
N_NODES = 10000
N_EDGES = 160000

_BE_TC = 640
_BN = 400

_NS = 16
_BE_SC = 128
_EPAD = 163840
_C0 = 102400
_C1 = _EPAD - _C0
_ACC_ROWS = N_NODES + 48


def _edge_kernel(geoT_ref, w1_ref, w2_ref, w3_ref, w4_ref,
                 we2_ref, m0_ref, m1_ref, m2_ref, m3_ref):
    geoTb = geoT_ref[0]
    rT = geoTb[0:1, :]
    nrow = (lax.broadcasted_iota(jnp.int32, (NUM_BESSEL, _BE_TC), 0)
            .astype(jnp.float32) + 1.0)
    arg = nrow * (np.pi / R_MAX) * rT
    pref = np.sqrt(2.0 / R_MAX)
    besselT = pref * jnp.sin(arg) / rT
    u = rT * (1.0 / R_MAX)
    u2 = u * u
    u4 = u2 * u2
    u6 = u4 * u2
    u7 = u6 * u
    u8 = u7 * u
    env = 1.0 - 28.0 * u6 + 48.0 * u7 - 21.0 * u8
    env = jnp.where(u < 1.0, env, 0.0)
    efT = besselT * env

    def _silu(x):
        return x / (1.0 + jnp.exp(-x))

    h = _silu(lax.dot_general(efT, w1_ref[...],
                              (((0,), (0,)), ((), ())),
                              preferred_element_type=jnp.float32))
    h = _silu(jnp.dot(h, w2_ref[...], preferred_element_type=jnp.float32))
    h = _silu(jnp.dot(h, w3_ref[...], preferred_element_type=jnp.float32))
    tp = jnp.dot(h, w4_ref[...], preferred_element_type=jnp.float32)

    gem = lax.dot_general(geoTb, jnp.eye(8, dtype=jnp.float32),
                          (((0,), (0,)), ((), ())),
                          preferred_element_type=jnp.float32)
    lane = lax.broadcasted_iota(jnp.int32, (_BE_TC, 8), 1).astype(jnp.float32)
    oh = (gem[:, 4:5] == lane).astype(jnp.float32)
    nfup = jnp.dot(oh, we2_ref[...],
                   preferred_element_type=jnp.float32)
    a = nfup * tp[:, :HIDDEN]
    b = nfup * tp[:, HIDDEN:]
    s3 = np.sqrt(3.0)
    m0_ref[...] = a
    m1_ref[...] = (s3 * gem[:, 1:2]) * b
    m2_ref[...] = (s3 * gem[:, 2:3]) * b
    m3_ref[...] = (s3 * gem[:, 3:4]) * b


def _edge_stage(geoT, W1, W2, W3, W4, We2p, off, echunk):
    grid = (echunk // _BE_TC,)
    eb = pl.BlockSpec((_BE_TC, HIDDEN), lambda i: (i, 0))
    outs = pl.pallas_call(
        _edge_kernel,
        grid=grid,
        in_specs=[
            pl.BlockSpec((1, 8, _BE_TC), lambda i: (i + off, 0, 0)),
            pl.BlockSpec((NUM_BESSEL, 64), lambda i: (0, 0)),
            pl.BlockSpec((64, 64), lambda i: (0, 0)),
            pl.BlockSpec((64, 64), lambda i: (0, 0)),
            pl.BlockSpec((64, 2 * HIDDEN), lambda i: (0, 0)),
            pl.BlockSpec((8, HIDDEN), lambda i: (0, 0)),
        ],
        out_specs=[eb, eb, eb, eb],
        out_shape=[jax.ShapeDtypeStruct((echunk, HIDDEN), jnp.float32)] * 4,
    )(geoT, W1, W2, W3, W4, We2p)
    return outs


_GB = 640
_NGB = N_EDGES // _GB
_NGBP = _EPAD // _GB
_NW = 32
_GIT = (_NGB + _NW - 1) // _NW


def _sc_gather_body(pos_ref, an_ref, snd_ref, rcv_ref, geoT_ref,
                    pos_v, an_v, sv, rv, gT, pos_sh, an_sh):
    cid = lax.axis_index("c")
    sid = lax.axis_index("s")
    w = sid * 2 + cid

    @pl.when(sid == 0)
    def _():
        pltpu.sync_copy(pos_ref, pos_sh)
        pltpu.sync_copy(an_ref, an_sh)
    plsc.subcore_barrier()
    pltpu.sync_copy(pos_sh, pos_v)
    pltpu.sync_copy(an_sh, an_v)

    def body(b, carry):
        blk = jnp.minimum(w + _NW * b, _NGB - 1)
        base = pl.multiple_of(blk * _GB, 128)
        pltpu.sync_copy(snd_ref.at[pl.ds(base, _GB)], sv)
        pltpu.sync_copy(rcv_ref.at[pl.ds(base, _GB)], rv)
        del base
        for k in range(_GB // 16):
            s16 = sv[pl.ds(k * 16, 16)]
            r16 = rv[pl.ds(k * 16, 16)]
            s4 = s16 * 4
            d4 = r16 * 4
            xs = plsc.load_gather(pos_v, [s4])
            ys = plsc.load_gather(pos_v, [s4 + 1])
            zs_ = plsc.load_gather(pos_v, [s4 + 2])
            xr = plsc.load_gather(pos_v, [d4])
            yr = plsc.load_gather(pos_v, [d4 + 1])
            zr = plsc.load_gather(pos_v, [d4 + 2])
            dx = xr - xs
            dy = yr - ys
            dz = zr - zs_
            r2 = dx * dx + dy * dy + dz * dz + 1e-9
            iy = jnp.int32(0x5F3759DF) - (
                lax.bitcast_convert_type(r2, jnp.int32) >> 1)
            y = lax.bitcast_convert_type(iy, jnp.float32)
            for _ in range(3):
                y = y * (1.5 - 0.5 * r2 * y * y)
            elem = plsc.load_gather(an_v, [s16]).astype(jnp.float32)
            gT[0, pl.ds(k * 16, 16)] = r2 * y
            gT[1, pl.ds(k * 16, 16)] = dx * y
            gT[2, pl.ds(k * 16, 16)] = dy * y
            gT[3, pl.ds(k * 16, 16)] = dz * y
            gT[4, pl.ds(k * 16, 16)] = elem
        pltpu.sync_copy(gT, geoT_ref.at[blk])
        return carry

    lax.fori_loop(0, _GIT, body, 0)


def _sc_gather(pos4, atomic_numbers, sender, receiver):
    mesh = plsc.VectorSubcoreMesh(core_axis_name="c", subcore_axis_name="s")
    f = pl.kernel(
        _sc_gather_body,
        out_type=jax.ShapeDtypeStruct((_NGBP, 8, _GB), jnp.float32),
        mesh=mesh,
        scratch_types=[
            pltpu.VMEM((4 * N_NODES,), jnp.float32),
            pltpu.VMEM((N_NODES,), jnp.int32),
            pltpu.VMEM((_GB,), jnp.int32),
            pltpu.VMEM((_GB,), jnp.int32),
            pltpu.VMEM((8, _GB), jnp.float32),
            pltpu.VMEM_SHARED((4 * N_NODES,), jnp.float32),
            pltpu.VMEM_SHARED((N_NODES,), jnp.int32),
        ],
        compiler_params=pltpu.CompilerParams(needs_layout_passes=False),
    )
    return f(pos4, atomic_numbers, sender, receiver)


def _sc_scatter_body(eps, nb, recv_ref, zeros_ref, m0, m1, m2, m3,
                     o0, o1, o2, o3,
                     idx_a, rows_a, idx_b, rows_b, acc,
                     sem_a, sem_b, sem_sa, sem_sb):
    cid = lax.axis_index("c")
    sid = lax.axis_index("s")

    def run_chunk(m_hbm, o_hbm):
        row = sid * 632

        @pl.when(sid < _NS - 1)
        def _():
            pltpu.sync_copy(zeros_ref.at[pl.ds(row, 632)],
                            acc.at[pl.ds(row, 632)])

        @pl.when(sid == _NS - 1)
        def _():
            pltpu.sync_copy(zeros_ref.at[pl.ds(row, _ACC_ROWS - 15 * 632)],
                            acc.at[pl.ds(row, _ACC_ROWS - 15 * 632)])
        plsc.subcore_barrier()

        def start_in(idx_p, rows_p, sem, b):
            base = pl.multiple_of(sid * eps + b * _BE_SC, 8)
            pltpu.async_copy(recv_ref.at[pl.ds(base, _BE_SC)], idx_p, sem)
            pltpu.async_copy(m_hbm.at[pl.ds(base, _BE_SC)], rows_p, sem)

        def wait_in(idx_p, rows_p, sem):
            pltpu.make_async_copy(
                recv_ref.at[pl.ds(0, _BE_SC)], idx_p, sem).wait()
            pltpu.make_async_copy(
                m_hbm.at[pl.ds(0, _BE_SC)], rows_p, sem).wait()

        npairs = (nb - 1) // 2
        start_in(idx_a, rows_a, sem_a, 0)

        def pbody(j, carry):
            wait_in(idx_a, rows_a, sem_a)
            start_in(idx_b, rows_b, sem_b, 2 * j + 1)
            sca = pltpu.async_copy(rows_a, acc.at[idx_a], sem_sa, add=True)
            wait_in(idx_b, rows_b, sem_b)
            sca.wait()
            start_in(idx_a, rows_a, sem_a, 2 * j + 2)
            scb = pltpu.async_copy(rows_b, acc.at[idx_b], sem_sb, add=True)
            scb.wait()
            return carry

        lax.fori_loop(0, npairs, pbody, 0)
        wait_in(idx_a, rows_a, sem_a)
        if nb % 2 == 0:
            start_in(idx_b, rows_b, sem_b, nb - 1)
            pltpu.sync_copy(rows_a, acc.at[idx_a], add=True)
            wait_in(idx_b, rows_b, sem_b)
            pltpu.sync_copy(rows_b, acc.at[idx_b], add=True)
        else:
            pltpu.sync_copy(rows_a, acc.at[idx_a], add=True)
        plsc.subcore_barrier()
        wrow = sid * 624

        @pl.when(sid < _NS - 1)
        def _():
            pltpu.sync_copy(acc.at[pl.ds(wrow, 624)],
                            o_hbm.at[pl.ds(wrow, 624)])

        @pl.when(sid == _NS - 1)
        def _():
            pltpu.sync_copy(acc.at[pl.ds(wrow, 640)],
                            o_hbm.at[pl.ds(wrow, 640)])
        plsc.subcore_barrier()

    @pl.when(cid == 0)
    def _():
        run_chunk(m0, o0)
        run_chunk(m1, o1)

    @pl.when(cid == 1)
    def _():
        run_chunk(m2, o2)
        run_chunk(m3, o3)


def _sc_scatter(recv, zeros_hbm, m0, m1, m2, m3, echunk):
    eps = echunk // _NS
    nb = eps // _BE_SC
    mesh = plsc.VectorSubcoreMesh(core_axis_name="c", subcore_axis_name="s")
    out_t = [jax.ShapeDtypeStruct((N_NODES, HIDDEN), jnp.float32)] * 4
    f = pl.kernel(
        functools.partial(_sc_scatter_body, eps, nb),
        out_type=out_t,
        mesh=mesh,
        scratch_types=[
            pltpu.VMEM((_BE_SC,), jnp.int32),
            pltpu.VMEM((_BE_SC, HIDDEN), jnp.float32),
            pltpu.VMEM((_BE_SC,), jnp.int32),
            pltpu.VMEM((_BE_SC, HIDDEN), jnp.float32),
            pltpu.VMEM_SHARED((_ACC_ROWS, HIDDEN), jnp.float32),
            pltpu.SemaphoreType.DMA,
            pltpu.SemaphoreType.DMA,
            pltpu.SemaphoreType.DMA,
            pltpu.SemaphoreType.DMA,
        ],
    )
    return f(recv, zeros_hbm, m0, m1, m2, m3)


def _node_kernel(m0_ref, m1_ref, m2_ref, m3_ref,
                 n0_ref, n1_ref, n2_ref, n3_ref, ohn_ref,
                 wp0_ref, wp1_ref, wp2_ref, wout_ref, out_ref):
    m0 = m0_ref[...] + n0_ref[...]
    m1 = m1_ref[...] + n1_ref[...]
    m2 = m2_ref[...] + n2_ref[...]
    m3 = m3_ref[...] + n3_ref[...]
    oh = ohn_ref[...]
    w0 = jnp.dot(oh, wp0_ref[...], preferred_element_type=jnp.float32)
    w1 = jnp.dot(oh, wp1_ref[...], preferred_element_type=jnp.float32)
    w2 = jnp.dot(oh, wp2_ref[...], preferred_element_type=jnp.float32)
    s1 = m0
    s2 = m0 * m0 + m1 * m1 + m2 * m2 + m3 * m3
    s3 = s1 * s2
    out_scalar = w0 * s1 + w1 * s2 + w2 * s3
    out_ref[...] = jnp.dot(out_scalar, wout_ref[...],
                           preferred_element_type=jnp.float32)


def _node_stage(ms, ns, onehot_n, Wp_pad, W_out):
    grid = (N_NODES // _BN,)
    nb = pl.BlockSpec((_BN, HIDDEN), lambda i: (i, 0))
    wb = pl.BlockSpec((8, HIDDEN), lambda i: (0, 0))
    return pl.pallas_call(
        _node_kernel,
        grid=grid,
        in_specs=[nb] * 8 + [
                  pl.BlockSpec((_BN, 8), lambda i: (i, 0)),
                  wb, wb, wb,
                  pl.BlockSpec((HIDDEN, HIDDEN), lambda i: (0, 0))],
        out_specs=nb,
        out_shape=jax.ShapeDtypeStruct((N_NODES, HIDDEN), jnp.float32),
    )(*ms, *ns, onehot_n, Wp_pad[0], Wp_pad[1], Wp_pad[2], W_out)


def kernel(positions, atomic_numbers, edge_index, W_emb, W_up,
           W1, W2, W3, W4, W_lin, Wp, W_out):
    sender = edge_index[0]
    receiver = edge_index[1]
    pos4 = jnp.pad(positions, ((0, 0), (0, 1))).reshape(-1)
    geoT = _sc_gather(pos4, atomic_numbers, sender, receiver)

    We2p = jnp.pad(W_emb @ W_up, ((0, 3), (0, 0)))
    W4 = jnp.concatenate([W4[:, 0::2], W4[:, 1::2]], axis=1)

    recv = jnp.concatenate([
        receiver.astype(jnp.int32),
        N_NODES + (jnp.arange(_EPAD - N_EDGES, dtype=jnp.int32) % 40),
    ])
    zeros_hbm = jnp.zeros((_ACC_ROWS, HIDDEN), jnp.float32)
    mc0 = _edge_stage(geoT, W1, W2, W3, W4, We2p, 0, _C0)
    sc0 = _sc_scatter(recv[:_C0], zeros_hbm, *mc0, echunk=_C0)
    mc1 = _edge_stage(geoT, W1, W2, W3, W4, We2p, _C0 // _BE_TC, _C1)
    sc1 = _sc_scatter(recv[_C0:], zeros_hbm, *mc1, echunk=_C1)

    onehot_n = jax.nn.one_hot(atomic_numbers, 8, dtype=jnp.float32)
    Wp_pad = jnp.pad(Wp, ((0, 0), (0, 3), (0, 0)))
    return _node_stage(sc0, sc1, onehot_n, Wp_pad, W_out)


# speedup vs baseline: 1.3867x; 1.0579x over previous
import functools

import jax
import jax.numpy as jnp
import numpy as np
from jax import lax
from jax.experimental import pallas as pl
from jax.experimental.pallas import tpu as pltpu
from jax.experimental.pallas import tpu_sc as plsc

R_MAX = 5.0
NUM_BESSEL = 8
HIDDEN = 128
N_NODES = 10000
N_EDGES = 160000

_BE_TC = 640
_BN = 400

_NS = 16
_BE_SC = 128
_EPAD = 163840
_C0 = 81920
_C1 = _EPAD - _C0
_ACC_ROWS = N_NODES + 48


def _edge_kernel(geoT_ref, w1_ref, w2_ref, w3_ref, w4_ref,
                 we2_ref, m0_ref, m1_ref, m2_ref, m3_ref):
    geoTb = geoT_ref[0]
    rT = geoTb[0:1, :]
    nrow = (lax.broadcasted_iota(jnp.int32, (NUM_BESSEL, _BE_TC), 0)
            .astype(jnp.float32) + 1.0)
    arg = nrow * (np.pi / R_MAX) * rT
    pref = np.sqrt(2.0 / R_MAX)
    besselT = pref * jnp.sin(arg) / rT
    u = rT * (1.0 / R_MAX)
    u2 = u * u
    u4 = u2 * u2
    u6 = u4 * u2
    u7 = u6 * u
    u8 = u7 * u
    env = 1.0 - 28.0 * u6 + 48.0 * u7 - 21.0 * u8
    env = jnp.where(u < 1.0, env, 0.0)
    efT = besselT * env

    def _silu(x):
        return x / (1.0 + jnp.exp(-x))

    h = _silu(lax.dot_general(efT, w1_ref[...],
                              (((0,), (0,)), ((), ())),
                              preferred_element_type=jnp.float32))
    h = _silu(jnp.dot(h, w2_ref[...], preferred_element_type=jnp.float32))
    h = _silu(jnp.dot(h, w3_ref[...], preferred_element_type=jnp.float32))
    tp = jnp.dot(h, w4_ref[...], preferred_element_type=jnp.float32)

    gem = lax.dot_general(geoTb, jnp.eye(8, dtype=jnp.float32),
                          (((0,), (0,)), ((), ())),
                          preferred_element_type=jnp.float32)
    lane = lax.broadcasted_iota(jnp.int32, (_BE_TC, 8), 1).astype(jnp.float32)
    oh = (gem[:, 4:5] == lane).astype(jnp.float32)
    nfup = jnp.dot(oh, we2_ref[...],
                   preferred_element_type=jnp.float32)
    a = nfup * tp[:, :HIDDEN]
    b = nfup * tp[:, HIDDEN:]
    s3 = np.sqrt(3.0)
    m0_ref[...] = a
    m1_ref[...] = (s3 * gem[:, 1:2]) * b
    m2_ref[...] = (s3 * gem[:, 2:3]) * b
    m3_ref[...] = (s3 * gem[:, 3:4]) * b


def _edge_stage(geoT, W1, W2, W3, W4, We2p, off, echunk):
    grid = (echunk // _BE_TC,)
    eb = pl.BlockSpec((_BE_TC, HIDDEN), lambda i: (i, 0))
    outs = pl.pallas_call(
        _edge_kernel,
        grid=grid,
        in_specs=[
            pl.BlockSpec((1, 8, _BE_TC), lambda i: (i + off, 0, 0)),
            pl.BlockSpec((NUM_BESSEL, 64), lambda i: (0, 0)),
            pl.BlockSpec((64, 64), lambda i: (0, 0)),
            pl.BlockSpec((64, 64), lambda i: (0, 0)),
            pl.BlockSpec((64, 2 * HIDDEN), lambda i: (0, 0)),
            pl.BlockSpec((8, HIDDEN), lambda i: (0, 0)),
        ],
        out_specs=[eb, eb, eb, eb],
        out_shape=[jax.ShapeDtypeStruct((echunk, HIDDEN), jnp.float32)] * 4,
    )(geoT, W1, W2, W3, W4, We2p)
    return outs


_GB = 640
_NGB = N_EDGES // _GB
_NGBP = _EPAD // _GB
_NW = 32
_GIT = (_NGB + _NW - 1) // _NW


def _sc_gather_body(pos_ref, an_ref, snd_ref, rcv_ref, geoT_ref,
                    pos_v, an_v, sv, rv, gT, pos_sh, an_sh):
    cid = lax.axis_index("c")
    sid = lax.axis_index("s")
    w = sid * 2 + cid

    @pl.when(sid == 0)
    def _():
        pltpu.sync_copy(pos_ref, pos_sh)
        pltpu.sync_copy(an_ref, an_sh)
    plsc.subcore_barrier()
    pltpu.sync_copy(pos_sh, pos_v)
    pltpu.sync_copy(an_sh, an_v)

    def body(b, carry):
        blk = jnp.minimum(w + _NW * b, _NGB - 1)
        base = pl.multiple_of(blk * _GB, 128)
        pltpu.sync_copy(snd_ref.at[pl.ds(base, _GB)], sv)
        pltpu.sync_copy(rcv_ref.at[pl.ds(base, _GB)], rv)
        del base
        for k in range(_GB // 16):
            s16 = sv[pl.ds(k * 16, 16)]
            r16 = rv[pl.ds(k * 16, 16)]
            s4 = s16 * 4
            d4 = r16 * 4
            xs = plsc.load_gather(pos_v, [s4])
            ys = plsc.load_gather(pos_v, [s4 + 1])
            zs_ = plsc.load_gather(pos_v, [s4 + 2])
            xr = plsc.load_gather(pos_v, [d4])
            yr = plsc.load_gather(pos_v, [d4 + 1])
            zr = plsc.load_gather(pos_v, [d4 + 2])
            dx = xr - xs
            dy = yr - ys
            dz = zr - zs_
            r2 = dx * dx + dy * dy + dz * dz + 1e-9
            iy = jnp.int32(0x5F3759DF) - (
                lax.bitcast_convert_type(r2, jnp.int32) >> 1)
            y = lax.bitcast_convert_type(iy, jnp.float32)
            for _ in range(3):
                y = y * (1.5 - 0.5 * r2 * y * y)
            elem = plsc.load_gather(an_v, [s16]).astype(jnp.float32)
            gT[0, pl.ds(k * 16, 16)] = r2 * y
            gT[1, pl.ds(k * 16, 16)] = dx * y
            gT[2, pl.ds(k * 16, 16)] = dy * y
            gT[3, pl.ds(k * 16, 16)] = dz * y
            gT[4, pl.ds(k * 16, 16)] = elem
        pltpu.sync_copy(gT, geoT_ref.at[blk])
        return carry

    lax.fori_loop(0, _GIT, body, 0)


def _sc_gather(pos4, atomic_numbers, sender, receiver):
    mesh = plsc.VectorSubcoreMesh(core_axis_name="c", subcore_axis_name="s")
    f = pl.kernel(
        _sc_gather_body,
        out_type=jax.ShapeDtypeStruct((_NGBP, 8, _GB), jnp.float32),
        mesh=mesh,
        scratch_types=[
            pltpu.VMEM((4 * N_NODES,), jnp.float32),
            pltpu.VMEM((N_NODES,), jnp.int32),
            pltpu.VMEM((_GB,), jnp.int32),
            pltpu.VMEM((_GB,), jnp.int32),
            pltpu.VMEM((8, _GB), jnp.float32),
            pltpu.VMEM_SHARED((4 * N_NODES,), jnp.float32),
            pltpu.VMEM_SHARED((N_NODES,), jnp.int32),
        ],
        compiler_params=pltpu.CompilerParams(needs_layout_passes=False),
    )
    return f(pos4, atomic_numbers, sender, receiver)


def _sc_scatter_body(eps, nb, recv_ref, zeros_ref, m0, m1, m2, m3,
                     o0, o1, o2, o3,
                     idx_a, rows_a, idx_b, rows_b, acc,
                     sem_a, sem_b, sem_sa, sem_sb):
    cid = lax.axis_index("c")
    sid = lax.axis_index("s")

    def run_chunk(m_hbm, o_hbm):
        row = sid * 632

        @pl.when(sid < _NS - 1)
        def _():
            pltpu.sync_copy(zeros_ref.at[pl.ds(row, 632)],
                            acc.at[pl.ds(row, 632)])

        @pl.when(sid == _NS - 1)
        def _():
            pltpu.sync_copy(zeros_ref.at[pl.ds(row, _ACC_ROWS - 15 * 632)],
                            acc.at[pl.ds(row, _ACC_ROWS - 15 * 632)])
        plsc.subcore_barrier()

        def start_in(idx_p, rows_p, sem, b):
            base = pl.multiple_of(sid * eps + b * _BE_SC, 8)
            pltpu.async_copy(recv_ref.at[pl.ds(base, _BE_SC)], idx_p, sem)
            pltpu.async_copy(m_hbm.at[pl.ds(base, _BE_SC)], rows_p, sem)

        def wait_in(idx_p, rows_p, sem):
            pltpu.make_async_copy(
                recv_ref.at[pl.ds(0, _BE_SC)], idx_p, sem).wait()
            pltpu.make_async_copy(
                m_hbm.at[pl.ds(0, _BE_SC)], rows_p, sem).wait()

        npairs = (nb - 1) // 2
        start_in(idx_a, rows_a, sem_a, 0)

        def pbody(j, carry):
            wait_in(idx_a, rows_a, sem_a)
            start_in(idx_b, rows_b, sem_b, 2 * j + 1)
            sca = pltpu.async_copy(rows_a, acc.at[idx_a], sem_sa, add=True)
            wait_in(idx_b, rows_b, sem_b)
            sca.wait()
            start_in(idx_a, rows_a, sem_a, 2 * j + 2)
            scb = pltpu.async_copy(rows_b, acc.at[idx_b], sem_sb, add=True)
            scb.wait()
            return carry

        lax.fori_loop(0, npairs, pbody, 0)
        wait_in(idx_a, rows_a, sem_a)
        if nb % 2 == 0:
            start_in(idx_b, rows_b, sem_b, nb - 1)
            pltpu.sync_copy(rows_a, acc.at[idx_a], add=True)
            wait_in(idx_b, rows_b, sem_b)
            pltpu.sync_copy(rows_b, acc.at[idx_b], add=True)
        else:
            pltpu.sync_copy(rows_a, acc.at[idx_a], add=True)
        plsc.subcore_barrier()
        wrow = sid * 624

        @pl.when(sid < _NS - 1)
        def _():
            pltpu.sync_copy(acc.at[pl.ds(wrow, 624)],
                            o_hbm.at[pl.ds(wrow, 624)])

        @pl.when(sid == _NS - 1)
        def _():
            pltpu.sync_copy(acc.at[pl.ds(wrow, 640)],
                            o_hbm.at[pl.ds(wrow, 640)])
        plsc.subcore_barrier()

    @pl.when(cid == 0)
    def _():
        run_chunk(m0, o0)
        run_chunk(m1, o1)

    @pl.when(cid == 1)
    def _():
        run_chunk(m2, o2)
        run_chunk(m3, o3)


def _sc_scatter(recv, zeros_hbm, m0, m1, m2, m3, echunk):
    eps = echunk // _NS
    nb = eps // _BE_SC
    mesh = plsc.VectorSubcoreMesh(core_axis_name="c", subcore_axis_name="s")
    out_t = [jax.ShapeDtypeStruct((N_NODES, HIDDEN), jnp.float32)] * 4
    f = pl.kernel(
        functools.partial(_sc_scatter_body, eps, nb),
        out_type=out_t,
        mesh=mesh,
        scratch_types=[
            pltpu.VMEM((_BE_SC,), jnp.int32),
            pltpu.VMEM((_BE_SC, HIDDEN), jnp.float32),
            pltpu.VMEM((_BE_SC,), jnp.int32),
            pltpu.VMEM((_BE_SC, HIDDEN), jnp.float32),
            pltpu.VMEM_SHARED((_ACC_ROWS, HIDDEN), jnp.float32),
            pltpu.SemaphoreType.DMA,
            pltpu.SemaphoreType.DMA,
            pltpu.SemaphoreType.DMA,
            pltpu.SemaphoreType.DMA,
        ],
    )
    return f(recv, zeros_hbm, m0, m1, m2, m3)


def _node_kernel(m0_ref, m1_ref, m2_ref, m3_ref,
                 n0_ref, n1_ref, n2_ref, n3_ref, ohn_ref,
                 wp0_ref, wp1_ref, wp2_ref, wout_ref, out_ref):
    m0 = m0_ref[...] + n0_ref[...]
    m1 = m1_ref[...] + n1_ref[...]
    m2 = m2_ref[...] + n2_ref[...]
    m3 = m3_ref[...] + n3_ref[...]
    oh = ohn_ref[...]
    w0 = jnp.dot(oh, wp0_ref[...], preferred_element_type=jnp.float32)
    w1 = jnp.dot(oh, wp1_ref[...], preferred_element_type=jnp.float32)
    w2 = jnp.dot(oh, wp2_ref[...], preferred_element_type=jnp.float32)
    s1 = m0
    s2 = m0 * m0 + m1 * m1 + m2 * m2 + m3 * m3
    s3 = s1 * s2
    out_scalar = w0 * s1 + w1 * s2 + w2 * s3
    out_ref[...] = jnp.dot(out_scalar, wout_ref[...],
                           preferred_element_type=jnp.float32)


def _node_stage(ms, ns, onehot_n, Wp_pad, W_out):
    grid = (N_NODES // _BN,)
    nb = pl.BlockSpec((_BN, HIDDEN), lambda i: (i, 0))
    wb = pl.BlockSpec((8, HIDDEN), lambda i: (0, 0))
    return pl.pallas_call(
        _node_kernel,
        grid=grid,
        in_specs=[nb] * 8 + [
                  pl.BlockSpec((_BN, 8), lambda i: (i, 0)),
                  wb, wb, wb,
                  pl.BlockSpec((HIDDEN, HIDDEN), lambda i: (0, 0))],
        out_specs=nb,
        out_shape=jax.ShapeDtypeStruct((N_NODES, HIDDEN), jnp.float32),
    )(*ms, *ns, onehot_n, Wp_pad[0], Wp_pad[1], Wp_pad[2], W_out)


def kernel(positions, atomic_numbers, edge_index, W_emb, W_up,
           W1, W2, W3, W4, W_lin, Wp, W_out):
    sender = edge_index[0]
    receiver = edge_index[1]
    pos4 = jnp.pad(positions, ((0, 0), (0, 1))).reshape(-1)
    geoT = _sc_gather(pos4, atomic_numbers, sender, receiver)

    We2p = jnp.pad(W_emb @ W_up, ((0, 3), (0, 0)))
    W4 = jnp.concatenate([W4[:, 0::2], W4[:, 1::2]], axis=1)

    recv = jnp.concatenate([
        receiver.astype(jnp.int32),
        N_NODES + (jnp.arange(_EPAD - N_EDGES, dtype=jnp.int32) % 40),
    ])
    zeros_hbm = jnp.zeros((_ACC_ROWS, HIDDEN), jnp.float32)
    mc0 = _edge_stage(geoT, W1, W2, W3, W4, We2p, 0, _C0)
    sc0 = _sc_scatter(recv[:_C0], zeros_hbm, *mc0, echunk=_C0)
    mc1 = _edge_stage(geoT, W1, W2, W3, W4, We2p, _C0 // _BE_TC, _C1)
    sc1 = _sc_scatter(recv[_C0:], zeros_hbm, *mc1, echunk=_C1)

    onehot_n = jax.nn.one_hot(atomic_numbers, 8, dtype=jnp.float32)
    Wp_pad = jnp.pad(Wp, ((0, 0), (0, 3), (0, 0)))
    return _node_stage(sc0, sc1, onehot_n, Wp_pad, W_out)


# speedup vs baseline: 1.4654x; 1.0567x over previous
import functools

import jax
import jax.numpy as jnp
import numpy as np
from jax import lax
from jax.experimental import pallas as pl
from jax.experimental.pallas import tpu as pltpu
from jax.experimental.pallas import tpu_sc as plsc

R_MAX = 5.0
NUM_BESSEL = 8
HIDDEN = 128
N_NODES = 10000
N_EDGES = 160000

_BE_TC = 640
_BN = 400

_NS = 16
_BE_SC = 128
_EPAD = 163840
_C0 = 81920
_C1 = _EPAD - _C0
_ACC_ROWS = N_NODES + 48


def _edge_kernel(geoT_ref, w1_ref, w2_ref, w3_ref, w4_ref,
                 we2_ref, m0_ref, m1_ref, m2_ref, m3_ref):
    geoTb = geoT_ref[0]
    rT = geoTb[0:1, :]
    nrow = (lax.broadcasted_iota(jnp.int32, (NUM_BESSEL, _BE_TC), 0)
            .astype(jnp.float32) + 1.0)
    arg = nrow * (np.pi / R_MAX) * rT
    pref = np.sqrt(2.0 / R_MAX)
    besselT = pref * jnp.sin(arg) / rT
    u = rT * (1.0 / R_MAX)
    u2 = u * u
    u4 = u2 * u2
    u6 = u4 * u2
    u7 = u6 * u
    u8 = u7 * u
    env = 1.0 - 28.0 * u6 + 48.0 * u7 - 21.0 * u8
    env = jnp.where(u < 1.0, env, 0.0)
    efT = besselT * env

    def _silu(x):
        return x / (1.0 + jnp.exp(-x))

    h = _silu(lax.dot_general(efT, w1_ref[...],
                              (((0,), (0,)), ((), ())),
                              preferred_element_type=jnp.float32))
    h = _silu(jnp.dot(h, w2_ref[...], preferred_element_type=jnp.float32))
    h = _silu(jnp.dot(h, w3_ref[...], preferred_element_type=jnp.float32))
    tp = jnp.dot(h, w4_ref[...], preferred_element_type=jnp.float32)

    gem = lax.dot_general(geoTb, jnp.eye(8, dtype=jnp.float32),
                          (((0,), (0,)), ((), ())),
                          preferred_element_type=jnp.float32)
    lane = lax.broadcasted_iota(jnp.int32, (_BE_TC, 8), 1).astype(jnp.float32)
    oh = (gem[:, 4:5] == lane).astype(jnp.float32)
    nfup = jnp.dot(oh, we2_ref[...],
                   preferred_element_type=jnp.float32)
    a = nfup * tp[:, :HIDDEN]
    b = nfup * tp[:, HIDDEN:]
    s3 = np.sqrt(3.0)
    m0_ref[...] = a
    m1_ref[...] = (s3 * gem[:, 1:2]) * b
    m2_ref[...] = (s3 * gem[:, 2:3]) * b
    m3_ref[...] = (s3 * gem[:, 3:4]) * b


def _edge_stage(geoT, W1, W2, W3, W4, We2p, off, echunk):
    grid = (echunk // _BE_TC,)
    eb = pl.BlockSpec((_BE_TC, HIDDEN), lambda i: (i, 0))
    outs = pl.pallas_call(
        _edge_kernel,
        grid=grid,
        in_specs=[
            pl.BlockSpec((1, 8, _BE_TC), lambda i: (i + off, 0, 0)),
            pl.BlockSpec((NUM_BESSEL, 64), lambda i: (0, 0)),
            pl.BlockSpec((64, 64), lambda i: (0, 0)),
            pl.BlockSpec((64, 64), lambda i: (0, 0)),
            pl.BlockSpec((64, 2 * HIDDEN), lambda i: (0, 0)),
            pl.BlockSpec((8, HIDDEN), lambda i: (0, 0)),
        ],
        out_specs=[eb, eb, eb, eb],
        out_shape=[jax.ShapeDtypeStruct((echunk, HIDDEN), jnp.float32)] * 4,
    )(geoT, W1, W2, W3, W4, We2p)
    return outs


_GB = 640
_NGB = N_EDGES // _GB
_NGBP = _EPAD // _GB
_NW = 32
_GIT = (_NGB + _NW - 1) // _NW


def _sc_gather_body(pos_ref, an_ref, snd_ref, rcv_ref, geoT_ref,
                    pos_v, an_v, sv, rv, gT, pos_sh, an_sh):
    cid = lax.axis_index("c")
    sid = lax.axis_index("s")
    w = sid * 2 + cid

    @pl.when(sid == 0)
    def _():
        pltpu.sync_copy(pos_ref, pos_sh)
        pltpu.sync_copy(an_ref, an_sh)
    plsc.subcore_barrier()
    pltpu.sync_copy(pos_sh, pos_v)
    pltpu.sync_copy(an_sh, an_v)

    def body(b, carry):
        blk = jnp.minimum(w + _NW * b, _NGB - 1)
        base = pl.multiple_of(blk * _GB, 128)
        pltpu.sync_copy(snd_ref.at[pl.ds(base, _GB)], sv)
        pltpu.sync_copy(rcv_ref.at[pl.ds(base, _GB)], rv)
        del base
        for k in range(_GB // 16):
            s16 = sv[pl.ds(k * 16, 16)]
            r16 = rv[pl.ds(k * 16, 16)]
            s4 = s16 * 4
            d4 = r16 * 4
            xs = plsc.load_gather(pos_v, [s4])
            ys = plsc.load_gather(pos_v, [s4 + 1])
            zs_ = plsc.load_gather(pos_v, [s4 + 2])
            xr = plsc.load_gather(pos_v, [d4])
            yr = plsc.load_gather(pos_v, [d4 + 1])
            zr = plsc.load_gather(pos_v, [d4 + 2])
            dx = xr - xs
            dy = yr - ys
            dz = zr - zs_
            r2 = dx * dx + dy * dy + dz * dz + 1e-9
            iy = jnp.int32(0x5F3759DF) - (
                lax.bitcast_convert_type(r2, jnp.int32) >> 1)
            y = lax.bitcast_convert_type(iy, jnp.float32)
            for _ in range(3):
                y = y * (1.5 - 0.5 * r2 * y * y)
            elem = plsc.load_gather(an_v, [s16]).astype(jnp.float32)
            gT[0, pl.ds(k * 16, 16)] = r2 * y
            gT[1, pl.ds(k * 16, 16)] = dx * y
            gT[2, pl.ds(k * 16, 16)] = dy * y
            gT[3, pl.ds(k * 16, 16)] = dz * y
            gT[4, pl.ds(k * 16, 16)] = elem
        pltpu.sync_copy(gT, geoT_ref.at[blk])
        return carry

    lax.fori_loop(0, _GIT, body, 0)


def _sc_gather(pos4, atomic_numbers, sender, receiver):
    mesh = plsc.VectorSubcoreMesh(core_axis_name="c", subcore_axis_name="s")
    f = pl.kernel(
        _sc_gather_body,
        out_type=jax.ShapeDtypeStruct((_NGBP, 8, _GB), jnp.float32),
        mesh=mesh,
        scratch_types=[
            pltpu.VMEM((4 * N_NODES,), jnp.float32),
            pltpu.VMEM((N_NODES,), jnp.int32),
            pltpu.VMEM((_GB,), jnp.int32),
            pltpu.VMEM((_GB,), jnp.int32),
            pltpu.VMEM((8, _GB), jnp.float32),
            pltpu.VMEM_SHARED((4 * N_NODES,), jnp.float32),
            pltpu.VMEM_SHARED((N_NODES,), jnp.int32),
        ],
        compiler_params=pltpu.CompilerParams(needs_layout_passes=False),
    )
    return f(pos4, atomic_numbers, sender, receiver)


def _sc_scatter_body(eps, nb, recv_ref, zeros_ref, m0, m1, m2, m3,
                     o0, o1, o2, o3,
                     idx_a, rows_a, idx_b, rows_b, idx_c, rows_c, acc,
                     sem_a, sem_b, sem_c, sem_sa, sem_sb, sem_sc):
    cid = lax.axis_index("c")
    sid = lax.axis_index("s")
    assert nb % 3 == 1

    def run_chunk(m_hbm, o_hbm):
        row = sid * 632

        @pl.when(sid < _NS - 1)
        def _():
            pltpu.sync_copy(zeros_ref.at[pl.ds(row, 632)],
                            acc.at[pl.ds(row, 632)])

        @pl.when(sid == _NS - 1)
        def _():
            pltpu.sync_copy(zeros_ref.at[pl.ds(row, _ACC_ROWS - 15 * 632)],
                            acc.at[pl.ds(row, _ACC_ROWS - 15 * 632)])
        plsc.subcore_barrier()

        def start_in(idx_p, rows_p, sem, b):
            base = pl.multiple_of(sid * eps + b * _BE_SC, 8)
            pltpu.async_copy(recv_ref.at[pl.ds(base, _BE_SC)], idx_p, sem)
            pltpu.async_copy(m_hbm.at[pl.ds(base, _BE_SC)], rows_p, sem)

        def wait_in(idx_p, rows_p, sem):
            pltpu.make_async_copy(
                recv_ref.at[pl.ds(0, _BE_SC)], idx_p, sem).wait()
            pltpu.make_async_copy(
                m_hbm.at[pl.ds(0, _BE_SC)], rows_p, sem).wait()

        start_in(idx_a, rows_a, sem_a, 0)
        start_in(idx_b, rows_b, sem_b, 1)

        def pbody(j, carry):
            start_in(idx_c, rows_c, sem_c, 3 * j + 2)
            wait_in(idx_a, rows_a, sem_a)
            sca = pltpu.async_copy(rows_a, acc.at[idx_a], sem_sa, add=True)
            sca.wait()
            start_in(idx_a, rows_a, sem_a, 3 * j + 3)
            wait_in(idx_b, rows_b, sem_b)
            scb = pltpu.async_copy(rows_b, acc.at[idx_b], sem_sb, add=True)
            scb.wait()
            start_in(idx_b, rows_b, sem_b, jnp.minimum(3 * j + 4, nb - 1))
            wait_in(idx_c, rows_c, sem_c)
            scc = pltpu.async_copy(rows_c, acc.at[idx_c], sem_sc, add=True)
            scc.wait()
            return carry

        lax.fori_loop(0, (nb - 1) // 3, pbody, 0)
        wait_in(idx_a, rows_a, sem_a)
        pltpu.sync_copy(rows_a, acc.at[idx_a], add=True)
        wait_in(idx_b, rows_b, sem_b)
        plsc.subcore_barrier()
        wrow = sid * 624

        @pl.when(sid < _NS - 1)
        def _():
            pltpu.sync_copy(acc.at[pl.ds(wrow, 624)],
                            o_hbm.at[pl.ds(wrow, 624)])

        @pl.when(sid == _NS - 1)
        def _():
            pltpu.sync_copy(acc.at[pl.ds(wrow, 640)],
                            o_hbm.at[pl.ds(wrow, 640)])
        plsc.subcore_barrier()

    @pl.when(cid == 0)
    def _():
        run_chunk(m0, o0)
        run_chunk(m1, o1)

    @pl.when(cid == 1)
    def _():
        run_chunk(m2, o2)
        run_chunk(m3, o3)


def _sc_scatter(recv, zeros_hbm, m0, m1, m2, m3, echunk):
    eps = echunk // _NS
    nb = eps // _BE_SC
    mesh = plsc.VectorSubcoreMesh(core_axis_name="c", subcore_axis_name="s")
    out_t = [jax.ShapeDtypeStruct((N_NODES, HIDDEN), jnp.float32)] * 4
    f = pl.kernel(
        functools.partial(_sc_scatter_body, eps, nb),
        out_type=out_t,
        mesh=mesh,
        scratch_types=[
            pltpu.VMEM((_BE_SC,), jnp.int32),
            pltpu.VMEM((_BE_SC, HIDDEN), jnp.float32),
            pltpu.VMEM((_BE_SC,), jnp.int32),
            pltpu.VMEM((_BE_SC, HIDDEN), jnp.float32),
            pltpu.VMEM((_BE_SC,), jnp.int32),
            pltpu.VMEM((_BE_SC, HIDDEN), jnp.float32),
            pltpu.VMEM_SHARED((_ACC_ROWS, HIDDEN), jnp.float32),
            pltpu.SemaphoreType.DMA,
            pltpu.SemaphoreType.DMA,
            pltpu.SemaphoreType.DMA,
            pltpu.SemaphoreType.DMA,
            pltpu.SemaphoreType.DMA,
            pltpu.SemaphoreType.DMA,
        ],
    )
    return f(recv, zeros_hbm, m0, m1, m2, m3)


def _node_kernel(m0_ref, m1_ref, m2_ref, m3_ref,
                 n0_ref, n1_ref, n2_ref, n3_ref, ohn_ref,
                 wp0_ref, wp1_ref, wp2_ref, wout_ref, out_ref):
    m0 = m0_ref[...] + n0_ref[...]
    m1 = m1_ref[...] + n1_ref[...]
    m2 = m2_ref[...] + n2_ref[...]
    m3 = m3_ref[...] + n3_ref[...]
    oh = ohn_ref[...]
    w0 = jnp.dot(oh, wp0_ref[...], preferred_element_type=jnp.float32)
    w1 = jnp.dot(oh, wp1_ref[...], preferred_element_type=jnp.float32)
    w2 = jnp.dot(oh, wp2_ref[...], preferred_element_type=jnp.float32)
    s1 = m0
    s2 = m0 * m0 + m1 * m1 + m2 * m2 + m3 * m3
    s3 = s1 * s2
    out_scalar = w0 * s1 + w1 * s2 + w2 * s3
    out_ref[...] = jnp.dot(out_scalar, wout_ref[...],
                           preferred_element_type=jnp.float32)


def _node_stage(ms, ns, onehot_n, Wp_pad, W_out):
    grid = (N_NODES // _BN,)
    nb = pl.BlockSpec((_BN, HIDDEN), lambda i: (i, 0))
    wb = pl.BlockSpec((8, HIDDEN), lambda i: (0, 0))
    return pl.pallas_call(
        _node_kernel,
        grid=grid,
        in_specs=[nb] * 8 + [
                  pl.BlockSpec((_BN, 8), lambda i: (i, 0)),
                  wb, wb, wb,
                  pl.BlockSpec((HIDDEN, HIDDEN), lambda i: (0, 0))],
        out_specs=nb,
        out_shape=jax.ShapeDtypeStruct((N_NODES, HIDDEN), jnp.float32),
    )(*ms, *ns, onehot_n, Wp_pad[0], Wp_pad[1], Wp_pad[2], W_out)


def kernel(positions, atomic_numbers, edge_index, W_emb, W_up,
           W1, W2, W3, W4, W_lin, Wp, W_out):
    sender = edge_index[0]
    receiver = edge_index[1]
    pos4 = jnp.pad(positions, ((0, 0), (0, 1))).reshape(-1)
    geoT = _sc_gather(pos4, atomic_numbers, sender, receiver)

    We2p = jnp.pad(W_emb @ W_up, ((0, 3), (0, 0)))
    W4 = jnp.concatenate([W4[:, 0::2], W4[:, 1::2]], axis=1)

    recv = jnp.concatenate([
        receiver.astype(jnp.int32),
        N_NODES + (jnp.arange(_EPAD - N_EDGES, dtype=jnp.int32) % 40),
    ])
    zeros_hbm = jnp.zeros((_ACC_ROWS, HIDDEN), jnp.float32)
    mc0 = _edge_stage(geoT, W1, W2, W3, W4, We2p, 0, _C0)
    sc0 = _sc_scatter(recv[:_C0], zeros_hbm, *mc0, echunk=_C0)
    mc1 = _edge_stage(geoT, W1, W2, W3, W4, We2p, _C0 // _BE_TC, _C1)
    sc1 = _sc_scatter(recv[_C0:], zeros_hbm, *mc1, echunk=_C1)

    onehot_n = jax.nn.one_hot(atomic_numbers, 8, dtype=jnp.float32)
    Wp_pad = jnp.pad(Wp, ((0, 0), (0, 3), (0, 0)))
    return _node_stage(sc0, sc1, onehot_n, Wp_pad, W_out)


# speedup vs baseline: 1.4671x; 1.0011x over previous
import functools

import jax
import jax.numpy as jnp
import numpy as np
from jax import lax
from jax.experimental import pallas as pl
from jax.experimental.pallas import tpu as pltpu
from jax.experimental.pallas import tpu_sc as plsc

R_MAX = 5.0
NUM_BESSEL = 8
HIDDEN = 128
N_NODES = 10000
N_EDGES = 160000

_BE_TC = 640
_BN = 400

_NS = 16
_BE_SC = 128
_EPAD = 163840
_C0 = 81920
_C1 = _EPAD - _C0
_ACC_ROWS = N_NODES + 48


def _edge_kernel(geoT_ref, w1_ref, w2_ref, w3_ref, w4_ref,
                 we2_ref, m0_ref, m1_ref, m2_ref, m3_ref):
    geoTb = geoT_ref[0]
    rT = geoTb[0:1, :]
    nrow = (lax.broadcasted_iota(jnp.int32, (NUM_BESSEL, _BE_TC), 0)
            .astype(jnp.float32) + 1.0)
    arg = nrow * (np.pi / R_MAX) * rT
    pref = np.sqrt(2.0 / R_MAX)
    besselT = pref * jnp.sin(arg) / rT
    u = rT * (1.0 / R_MAX)
    u2 = u * u
    u4 = u2 * u2
    u6 = u4 * u2
    u7 = u6 * u
    u8 = u7 * u
    env = 1.0 - 28.0 * u6 + 48.0 * u7 - 21.0 * u8
    env = jnp.where(u < 1.0, env, 0.0)
    efT = besselT * env

    def _silu(x):
        return x / (1.0 + jnp.exp(-x))

    h = _silu(lax.dot_general(efT, w1_ref[...],
                              (((0,), (0,)), ((), ())),
                              preferred_element_type=jnp.float32))
    h = _silu(jnp.dot(h, w2_ref[...], preferred_element_type=jnp.float32))
    h = _silu(jnp.dot(h, w3_ref[...], preferred_element_type=jnp.float32))
    tp = jnp.dot(h, w4_ref[...], preferred_element_type=jnp.float32)

    gem = lax.dot_general(geoTb, jnp.eye(8, dtype=jnp.float32),
                          (((0,), (0,)), ((), ())),
                          preferred_element_type=jnp.float32)
    lane = lax.broadcasted_iota(jnp.int32, (_BE_TC, 8), 1).astype(jnp.float32)
    oh = (gem[:, 4:5] == lane).astype(jnp.float32)
    nfup = jnp.dot(oh, we2_ref[...],
                   preferred_element_type=jnp.float32)
    a = nfup * tp[:, :HIDDEN]
    b = nfup * tp[:, HIDDEN:]
    s3 = np.sqrt(3.0)
    m0_ref[...] = a
    m1_ref[...] = (s3 * gem[:, 1:2]) * b
    m2_ref[...] = (s3 * gem[:, 2:3]) * b
    m3_ref[...] = (s3 * gem[:, 3:4]) * b


def _edge_stage(geoT, W1, W2, W3, W4, We2p, off, echunk):
    grid = (echunk // _BE_TC,)
    eb = pl.BlockSpec((_BE_TC, HIDDEN), lambda i: (i, 0))
    outs = pl.pallas_call(
        _edge_kernel,
        grid=grid,
        in_specs=[
            pl.BlockSpec((1, 8, _BE_TC), lambda i: (i + off, 0, 0)),
            pl.BlockSpec((NUM_BESSEL, 64), lambda i: (0, 0)),
            pl.BlockSpec((64, 64), lambda i: (0, 0)),
            pl.BlockSpec((64, 64), lambda i: (0, 0)),
            pl.BlockSpec((64, 2 * HIDDEN), lambda i: (0, 0)),
            pl.BlockSpec((8, HIDDEN), lambda i: (0, 0)),
        ],
        out_specs=[eb, eb, eb, eb],
        out_shape=[jax.ShapeDtypeStruct((echunk, HIDDEN), jnp.float32)] * 4,
    )(geoT, W1, W2, W3, W4, We2p)
    return outs


_GB = 640
_NGB = N_EDGES // _GB
_NGBP = _EPAD // _GB
_NW = 32
_GIT = (_NGB + _NW - 1) // _NW


def _sc_gather_body(pos_ref, an_ref, snd_ref, rcv_ref, geoT_ref,
                    pos_v, an_v, sv, rv, gT, pos_sh, an_sh):
    cid = lax.axis_index("c")
    sid = lax.axis_index("s")
    w = sid * 2 + cid

    @pl.when(sid == 0)
    def _():
        pltpu.sync_copy(pos_ref, pos_sh)
        pltpu.sync_copy(an_ref, an_sh)
    plsc.subcore_barrier()
    pltpu.sync_copy(pos_sh, pos_v)
    pltpu.sync_copy(an_sh, an_v)

    def body(b, carry):
        blk = jnp.minimum(w + _NW * b, _NGB - 1)
        base = pl.multiple_of(blk * _GB, 128)
        pltpu.sync_copy(snd_ref.at[pl.ds(base, _GB)], sv)
        pltpu.sync_copy(rcv_ref.at[pl.ds(base, _GB)], rv)
        for k in range(_GB // 16):
            s16 = sv[pl.ds(k * 16, 16)]
            r16 = rv[pl.ds(k * 16, 16)]
            s4 = s16 * 4
            d4 = r16 * 4
            xs = plsc.load_gather(pos_v, [s4])
            ys = plsc.load_gather(pos_v, [s4 + 1])
            zs_ = plsc.load_gather(pos_v, [s4 + 2])
            xr = plsc.load_gather(pos_v, [d4])
            yr = plsc.load_gather(pos_v, [d4 + 1])
            zr = plsc.load_gather(pos_v, [d4 + 2])
            dx = xr - xs
            dy = yr - ys
            dz = zr - zs_
            r2 = dx * dx + dy * dy + dz * dz + 1e-9
            iy = jnp.int32(0x5F3759DF) - (
                lax.bitcast_convert_type(r2, jnp.int32) >> 1)
            y = lax.bitcast_convert_type(iy, jnp.float32)
            for _ in range(3):
                y = y * (1.5 - 0.5 * r2 * y * y)
            elem = plsc.load_gather(an_v, [s16]).astype(jnp.float32)
            gT[0, pl.ds(k * 16, 16)] = r2 * y
            gT[1, pl.ds(k * 16, 16)] = dx * y
            gT[2, pl.ds(k * 16, 16)] = dy * y
            gT[3, pl.ds(k * 16, 16)] = dz * y
            gT[4, pl.ds(k * 16, 16)] = elem
        pltpu.sync_copy(gT, geoT_ref.at[blk])
        return carry

    lax.fori_loop(0, _GIT, body, 0)


def _sc_gather(pos4, atomic_numbers, sender, receiver):
    mesh = plsc.VectorSubcoreMesh(core_axis_name="c", subcore_axis_name="s")
    f = pl.kernel(
        _sc_gather_body,
        out_type=jax.ShapeDtypeStruct((_NGBP, 8, _GB), jnp.float32),
        mesh=mesh,
        scratch_types=[
            pltpu.VMEM((4 * N_NODES,), jnp.float32),
            pltpu.VMEM((N_NODES,), jnp.int32),
            pltpu.VMEM((_GB,), jnp.int32),
            pltpu.VMEM((_GB,), jnp.int32),
            pltpu.VMEM((8, _GB), jnp.float32),
            pltpu.VMEM_SHARED((4 * N_NODES,), jnp.float32),
            pltpu.VMEM_SHARED((N_NODES,), jnp.int32),
        ],
        compiler_params=pltpu.CompilerParams(needs_layout_passes=False),
    )
    return f(pos4, atomic_numbers, sender, receiver)


def _sc_scatter_body(eps, nb, recv_ref, zeros_ref, m0, m1, m2, m3,
                     o0, o1, o2, o3,
                     idx_a, rows_a, idx_b, rows_b, idx_c, rows_c, acc,
                     sem_a, sem_b, sem_c, sem_sa, sem_sb, sem_sc):
    cid = lax.axis_index("c")
    sid = lax.axis_index("s")
    assert nb % 3 == 1

    def run_chunk(m_hbm, o_hbm):
        row = sid * 632

        @pl.when(sid < _NS - 1)
        def _():
            pltpu.sync_copy(zeros_ref.at[pl.ds(row, 632)],
                            acc.at[pl.ds(row, 632)])

        @pl.when(sid == _NS - 1)
        def _():
            pltpu.sync_copy(zeros_ref.at[pl.ds(row, _ACC_ROWS - 15 * 632)],
                            acc.at[pl.ds(row, _ACC_ROWS - 15 * 632)])
        plsc.subcore_barrier()

        def start_in(idx_p, rows_p, sem, b):
            base = pl.multiple_of(sid * eps + b * _BE_SC, 8)
            pltpu.async_copy(recv_ref.at[pl.ds(base, _BE_SC)], idx_p, sem)
            pltpu.async_copy(m_hbm.at[pl.ds(base, _BE_SC)], rows_p, sem)

        def wait_in(idx_p, rows_p, sem):
            pltpu.make_async_copy(
                recv_ref.at[pl.ds(0, _BE_SC)], idx_p, sem).wait()
            pltpu.make_async_copy(
                m_hbm.at[pl.ds(0, _BE_SC)], rows_p, sem).wait()

        start_in(idx_a, rows_a, sem_a, 0)
        start_in(idx_b, rows_b, sem_b, 1)

        def pbody(j, carry):
            start_in(idx_c, rows_c, sem_c, 3 * j + 2)
            wait_in(idx_a, rows_a, sem_a)
            sca = pltpu.async_copy(rows_a, acc.at[idx_a], sem_sa, add=True)
            sca.wait()
            start_in(idx_a, rows_a, sem_a, 3 * j + 3)
            wait_in(idx_b, rows_b, sem_b)
            scb = pltpu.async_copy(rows_b, acc.at[idx_b], sem_sb, add=True)
            scb.wait()
            start_in(idx_b, rows_b, sem_b, jnp.minimum(3 * j + 4, nb - 1))
            wait_in(idx_c, rows_c, sem_c)
            scc = pltpu.async_copy(rows_c, acc.at[idx_c], sem_sc, add=True)
            scc.wait()
            return carry

        lax.fori_loop(0, (nb - 1) // 3, pbody, 0)
        wait_in(idx_a, rows_a, sem_a)
        pltpu.sync_copy(rows_a, acc.at[idx_a], add=True)
        wait_in(idx_b, rows_b, sem_b)
        plsc.subcore_barrier()
        wrow = sid * 624

        @pl.when(sid < _NS - 1)
        def _():
            pltpu.sync_copy(acc.at[pl.ds(wrow, 624)],
                            o_hbm.at[pl.ds(wrow, 624)])

        @pl.when(sid == _NS - 1)
        def _():
            pltpu.sync_copy(acc.at[pl.ds(wrow, 640)],
                            o_hbm.at[pl.ds(wrow, 640)])
        plsc.subcore_barrier()

    @pl.when(cid == 0)
    def _():
        run_chunk(m0, o0)
        run_chunk(m1, o1)

    @pl.when(cid == 1)
    def _():
        run_chunk(m2, o2)
        run_chunk(m3, o3)


def _sc_scatter(recv, zeros_hbm, m0, m1, m2, m3, echunk):
    eps = echunk // _NS
    nb = eps // _BE_SC
    mesh = plsc.VectorSubcoreMesh(core_axis_name="c", subcore_axis_name="s")
    out_t = [jax.ShapeDtypeStruct((N_NODES, HIDDEN), jnp.float32)] * 4
    f = pl.kernel(
        functools.partial(_sc_scatter_body, eps, nb),
        out_type=out_t,
        mesh=mesh,
        scratch_types=[
            pltpu.VMEM((_BE_SC,), jnp.int32),
            pltpu.VMEM((_BE_SC, HIDDEN), jnp.float32),
            pltpu.VMEM((_BE_SC,), jnp.int32),
            pltpu.VMEM((_BE_SC, HIDDEN), jnp.float32),
            pltpu.VMEM((_BE_SC,), jnp.int32),
            pltpu.VMEM((_BE_SC, HIDDEN), jnp.float32),
            pltpu.VMEM_SHARED((_ACC_ROWS, HIDDEN), jnp.float32),
            pltpu.SemaphoreType.DMA,
            pltpu.SemaphoreType.DMA,
            pltpu.SemaphoreType.DMA,
            pltpu.SemaphoreType.DMA,
            pltpu.SemaphoreType.DMA,
            pltpu.SemaphoreType.DMA,
        ],
    )
    return f(recv, zeros_hbm, m0, m1, m2, m3)


def _node_kernel(m0_ref, m1_ref, m2_ref, m3_ref,
                 n0_ref, n1_ref, n2_ref, n3_ref, ohn_ref,
                 wp0_ref, wp1_ref, wp2_ref, wout_ref, out_ref):
    m0 = m0_ref[...] + n0_ref[...]
    m1 = m1_ref[...] + n1_ref[...]
    m2 = m2_ref[...] + n2_ref[...]
    m3 = m3_ref[...] + n3_ref[...]
    oh = ohn_ref[...]
    w0 = jnp.dot(oh, wp0_ref[...], preferred_element_type=jnp.float32)
    w1 = jnp.dot(oh, wp1_ref[...], preferred_element_type=jnp.float32)
    w2 = jnp.dot(oh, wp2_ref[...], preferred_element_type=jnp.float32)
    s1 = m0
    s2 = m0 * m0 + m1 * m1 + m2 * m2 + m3 * m3
    s3 = s1 * s2
    out_scalar = w0 * s1 + w1 * s2 + w2 * s3
    out_ref[...] = jnp.dot(out_scalar, wout_ref[...],
                           preferred_element_type=jnp.float32)


def _node_stage(ms, ns, onehot_n, Wp_pad, W_out):
    grid = (N_NODES // _BN,)
    nb = pl.BlockSpec((_BN, HIDDEN), lambda i: (i, 0))
    wb = pl.BlockSpec((8, HIDDEN), lambda i: (0, 0))
    return pl.pallas_call(
        _node_kernel,
        grid=grid,
        in_specs=[nb] * 8 + [
                  pl.BlockSpec((_BN, 8), lambda i: (i, 0)),
                  wb, wb, wb,
                  pl.BlockSpec((HIDDEN, HIDDEN), lambda i: (0, 0))],
        out_specs=nb,
        out_shape=jax.ShapeDtypeStruct((N_NODES, HIDDEN), jnp.float32),
    )(*ms, *ns, onehot_n, Wp_pad[0], Wp_pad[1], Wp_pad[2], W_out)


def kernel(positions, atomic_numbers, edge_index, W_emb, W_up,
           W1, W2, W3, W4, W_lin, Wp, W_out):
    sender = edge_index[0]
    receiver = edge_index[1]
    pos4 = jnp.pad(positions, ((0, 0), (0, 1))).reshape(-1)
    geoT = _sc_gather(pos4, atomic_numbers, sender, receiver)

    We2p = jnp.pad(W_emb @ W_up, ((0, 3), (0, 0)))
    W4 = jnp.concatenate([W4[:, 0::2], W4[:, 1::2]], axis=1)

    recv = jnp.concatenate([
        receiver.astype(jnp.int32),
        N_NODES + (jnp.arange(_EPAD - N_EDGES, dtype=jnp.int32) % 40),
    ])
    zeros_hbm = jnp.zeros((_ACC_ROWS, HIDDEN), jnp.float32)
    mc0 = _edge_stage(geoT, W1, W2, W3, W4, We2p, 0, _C0)
    sc0 = _sc_scatter(recv[:_C0], zeros_hbm, *mc0, echunk=_C0)
    mc1 = _edge_stage(geoT, W1, W2, W3, W4, We2p, _C0 // _BE_TC, _C1)
    sc1 = _sc_scatter(recv[_C0:], zeros_hbm, *mc1, echunk=_C1)

    onehot_n = jax.nn.one_hot(atomic_numbers, 8, dtype=jnp.float32)
    Wp_pad = jnp.pad(Wp, ((0, 0), (0, 3), (0, 0)))
    return _node_stage(sc0, sc1, onehot_n, Wp_pad, W_out)
